# baseline (speedup 1.0000x reference)
import functools

import jax
import jax.numpy as jnp
from jax import lax
from jax.experimental import pallas as pl
from jax.experimental.pallas import tpu as pltpu
from jax.experimental.pallas import tpu_sc as plsc

_B = 4096
_L = 200
_D = 512
_CH = 40
_NCH = _L // _CH
_NC = 2
_NS = 16
_NW = _NC * _NS
_BPW = _B // _NW
_LANES = 16


def _body(idx_hbm, pos_hbm, table_hbm, out_hbm, idx_v, pos_v, gbuf0, gbuf1,
          obuf0, obuf1, gsem0, gsem1, wsem0, wsem1):
    c = lax.axis_index("c")
    s = lax.axis_index("s")
    wid = s * _NC + c
    base = wid * _BPW
    gbufs = (gbuf0, gbuf1)
    obufs = (obuf0, obuf1)
    gsems = (gsem0, gsem1)
    wsems = (wsem0, wsem1)

    def start_gather(bl, p):
        pltpu.async_copy(
            table_hbm.at[idx_v.at[pl.ds(bl * _CH, _CH)]], gbufs[p], gsems[p])

    def wait_gather(bl, p):
        pltpu.make_async_copy(
            table_hbm.at[idx_v.at[pl.ds(bl * _CH, _CH)]], gbufs[p],
            gsems[p]).wait()

    def out_slice(bl, ch):
        row0 = (base + bl) * _L + ch * _CH
        return out_hbm.at[pl.ds(row0, _CH)]

    def convert_add(p):
        gbuf, obuf = gbufs[p], obufs[p]
        shift = jnp.full((_LANES,), 16, dtype=jnp.int32)
        himask = jnp.full((_LANES,), -65536, dtype=jnp.int32)

        def r_body(r, _):
            for jj in range(_D // (2 * _LANES)):
                u = gbuf[r, pl.ds(jj * _LANES, _LANES)]
                lo = u.astype(jnp.float32)
                hi = u.astype(jnp.float32)
                sl0 = pl.ds(jj * 2 * _LANES, _LANES)
                sl1 = pl.ds(jj * 2 * _LANES + _LANES, _LANES)
                obuf[r, sl0] = lo + pos_v[r, sl0]
                obuf[r, sl1] = hi + pos_v[r, sl1]
            return 0

        lax.fori_loop(0, _CH, r_body, 0)

    def start_write(bl, p, ch):
        pltpu.async_copy(obufs[p], out_slice(bl, ch), wsems[p])

    def wait_write(bl, p, ch):
        pltpu.make_async_copy(obufs[p], out_slice(bl, ch), wsems[p]).wait()

    for ch in range(_NCH):
        pltpu.sync_copy(
            idx_hbm.at[pl.ds(ch * _B * _CH + base * _CH, _BPW * _CH)], idx_v)
        pltpu.sync_copy(pos_hbm.at[pl.ds(ch * _CH, _CH)], pos_v)

        start_gather(0, 0)
        wait_gather(0, 0)
        start_gather(1, 1)
        convert_add(0)
        start_write(0, 0, ch)
        wait_gather(1, 1)
        start_gather(2, 0)
        convert_add(1)
        start_write(1, 1, ch)

        def pair_body(i, _):
            for k in (0, 1):
                bl = 2 * i + 2 + k
                p = k
                o = 1 - k
                wait_gather(bl, p)

                @pl.when(bl < _BPW - 1)
                def _():
                    start_gather(bl + 1, o)

                wait_write(bl - 2, p, ch)
                convert_add(p)
                start_write(bl, p, ch)
            return 0

        lax.fori_loop(0, (_BPW - 2) // 2, pair_body, 0)

        wait_write(_BPW - 2, 0, ch)
        wait_write(_BPW - 1, 1, ch)


@jax.jit
def kernel(inputs, table, pos_embed):
    idx_r = (inputs.astype(jnp.int32)
             .reshape(_B, _NCH, _CH)
             .transpose(1, 0, 2)
             .reshape(_NCH * _B * _CH))
    bits = lax.bitcast_convert_type(table, jnp.uint32) + jnp.uint32(0x8000)
    b3 = bits.reshape(-1, _D // 32, 2, _LANES)
    packed = (b3[:, :, 0, :] >> 16) | (b3[:, :, 1, :] & jnp.uint32(0xFFFF0000))
    tb = lax.bitcast_convert_type(packed.reshape(-1, _D // 2), jnp.int32)
    pos2 = pos_embed.reshape(_L, _D)
    mesh = plsc.VectorSubcoreMesh(core_axis_name="c", subcore_axis_name="s")
    run = pl.kernel(
        _body,
        out_type=jax.ShapeDtypeStruct((_B * _L, _D), jnp.float32),
        mesh=mesh,
        scratch_types=[
            pltpu.VMEM((_BPW * _CH,), jnp.int32),
            pltpu.VMEM((_CH, _D), jnp.float32),
            pltpu.VMEM((_CH, _D // 2), jnp.int32),
            pltpu.VMEM((_CH, _D // 2), jnp.int32),
            pltpu.VMEM((_CH, _D), jnp.float32),
            pltpu.VMEM((_CH, _D), jnp.float32),
            pltpu.SemaphoreType.DMA,
            pltpu.SemaphoreType.DMA,
            pltpu.SemaphoreType.DMA,
            pltpu.SemaphoreType.DMA,
        ],
    )
    out = run(idx_r, pos2, tb)
    return out.reshape(_B, _L, _D)


# speedup vs baseline: 2.0183x; 2.0183x over previous
import functools

import jax
import jax.numpy as jnp
from jax import lax
from jax.experimental import pallas as pl
from jax.experimental.pallas import tpu as pltpu
from jax.experimental.pallas import tpu_sc as plsc

_B = 4096
_L = 200
_D = 512
_CH = 40
_NCH = _L // _CH
_NC = 2
_NS = 16
_NW = _NC * _NS
_BPW = _B // _NW
_LANES = 16
_NBUF = 4


def _body(idx_hbm, pos_hbm, table_hbm, out_hbm, idx_v, pos_v,
          buf0, buf1, buf2, buf3,
          gsem0, gsem1, gsem2, gsem3, wsem0, wsem1, wsem2, wsem3):
    c = lax.axis_index("c")
    s = lax.axis_index("s")
    wid = s * _NC + c
    base = wid * _BPW
    bufs = (buf0, buf1, buf2, buf3)
    gsems = (gsem0, gsem1, gsem2, gsem3)
    wsems = (wsem0, wsem1, wsem2, wsem3)

    def start_gather(bl, p):
        pltpu.async_copy(
            table_hbm.at[idx_v.at[pl.ds(bl * _CH, _CH)]], bufs[p], gsems[p])

    def wait_gather(bl, p):
        pltpu.make_async_copy(
            table_hbm.at[idx_v.at[pl.ds(bl * _CH, _CH)]], bufs[p],
            gsems[p]).wait()

    def out_slice(bl, ch):
        row0 = (base + bl) * _L + ch * _CH
        return out_hbm.at[pl.ds(row0, _CH)]

    def add(p):
        buf = bufs[p]

        def r_body(r, _):
            for jj in range(_D // _LANES):
                sl = pl.ds(jj * _LANES, _LANES)
                buf[r, sl] = buf[r, sl] + pos_v[r, sl]
            return 0

        lax.fori_loop(0, _CH, r_body, 0)

    def start_write(bl, p, ch):
        pltpu.async_copy(bufs[p], out_slice(bl, ch), wsems[p])

    def wait_write(bl, p, ch):
        pltpu.make_async_copy(bufs[p], out_slice(bl, ch), wsems[p]).wait()

    for ch in range(_NCH):
        pltpu.sync_copy(
            idx_hbm.at[pl.ds(ch * _B * _CH + base * _CH, _BPW * _CH)], idx_v)
        pltpu.sync_copy(pos_hbm.at[pl.ds(ch * _CH, _CH)], pos_v)

        start_gather(0, 0)
        start_gather(1, 1)
        start_gather(2, 2)

        wait_gather(0, 0)
        start_gather(3, 3)
        add(0)
        start_write(0, 0, ch)

        def quad_body(i, _):
            for k in range(_NBUF):
                bl = _NBUF * i + 1 + k
                p = (1 + k) % _NBUF
                q = (p + 3) % _NBUF
                wait_gather(bl, p)
                wait_write(bl - 1, q, ch)
                start_gather(bl + 3, q)
                add(p)
                start_write(bl, p, ch)
            return 0

        lax.fori_loop(0, (_BPW - _NBUF) // _NBUF, quad_body, 0)

        for bl in range(_BPW - 3, _BPW):
            p = bl % _NBUF
            wait_gather(bl, p)
            add(p)
            start_write(bl, p, ch)

        for bl in range(_BPW - _NBUF, _BPW):
            wait_write(bl, bl % _NBUF, ch)


@jax.jit
def kernel(inputs, table, pos_embed):
    idx_r = (inputs.astype(jnp.int32)
             .reshape(_B, _NCH, _CH)
             .transpose(1, 0, 2)
             .reshape(_NCH * _B * _CH))
    pos2 = pos_embed.reshape(_L, _D)
    mesh = plsc.VectorSubcoreMesh(core_axis_name="c", subcore_axis_name="s")
    run = pl.kernel(
        _body,
        out_type=jax.ShapeDtypeStruct((_B * _L, _D), jnp.float32),
        mesh=mesh,
        scratch_types=(
            [pltpu.VMEM((_BPW * _CH,), jnp.int32),
             pltpu.VMEM((_CH, _D), jnp.float32)]
            + [pltpu.VMEM((_CH, _D), jnp.float32) for _ in range(_NBUF)]
            + [pltpu.SemaphoreType.DMA] * (2 * _NBUF)
        ),
    )
    out = run(idx_r, pos2, table)
    return out.reshape(_B, _L, _D)
